# baseline (speedup 1.0000x reference)
import jax
import jax.numpy as jnp
from jax import lax
from jax.experimental import pallas as pl
from jax.experimental.pallas import tpu as pltpu
from jax.experimental.pallas import tpu_sc as plsc

_EMB = 32
_BATCH = 16384
_NUM_CORES = 2
_NUM_SUBCORES = 16
_LANES = 16
_NW = _NUM_CORES * _NUM_SUBCORES
_BPW = _BATCH // _NW
_GROUPS = _BPW // _LANES


def _tec_body(h_idx, r_idx, t_idx, time_idx, ent, rel, tim, out,
              hi_v, ri_v, ti_v, mi_v, h_v, r_v, t_v, m_v, o_v, sem):
  wid = lax.axis_index("s") * _NUM_CORES + lax.axis_index("c")
  base = wid * _BPW

  pltpu.sync_copy(h_idx.at[pl.ds(base, _BPW)], hi_v)
  pltpu.sync_copy(r_idx.at[pl.ds(base, _BPW)], ri_v)
  pltpu.sync_copy(t_idx.at[pl.ds(base, _BPW)], ti_v)
  pltpu.sync_copy(time_idx.at[pl.ds(base, _BPW)], mi_v)

  c1 = pltpu.async_copy(ent.at[hi_v], h_v, sem)
  c2 = pltpu.async_copy(ent.at[ti_v], t_v, sem)
  c3 = pltpu.async_copy(rel.at[ri_v], r_v, sem)
  c4 = pltpu.async_copy(tim.at[mi_v], m_v, sem)
  c1.wait()
  c2.wait()
  c3.wait()
  c4.wait()

  lanes = lax.iota(jnp.int32, _LANES)

  def group(g, carry):
    res = jnp.zeros((_LANES,), jnp.float32)
    for j in range(_LANES):
      e = g * _LANES + j
      h0 = h_v[e, pl.ds(0, _LANES)]
      h1 = h_v[e, pl.ds(_LANES, _LANES)]
      r0 = r_v[e, pl.ds(0, _LANES)]
      r1 = r_v[e, pl.ds(_LANES, _LANES)]
      t0 = t_v[e, pl.ds(0, _LANES)]
      t1 = t_v[e, pl.ds(_LANES, _LANES)]
      m0 = m_v[e, pl.ds(0, _LANES)]
      m1 = m_v[e, pl.ds(_LANES, _LANES)]
      s = jnp.abs(h0 + r0 + m0 - t0) + jnp.abs(h1 + r1 + m1 - t1)
      res = jnp.where(lanes == j, -jnp.sum(s), res)
    o_v[pl.ds(g * _LANES, _LANES)] = res
    return carry

  lax.fori_loop(0, _GROUPS, group, 0)
  pltpu.sync_copy(o_v, out.at[pl.ds(base, _BPW)])


_mesh = plsc.VectorSubcoreMesh(
    core_axis_name="c", subcore_axis_name="s",
    num_cores=_NUM_CORES, num_subcores=_NUM_SUBCORES)

_sc_call = pl.kernel(
    _tec_body,
    out_type=jax.ShapeDtypeStruct((_BATCH,), jnp.float32),
    mesh=_mesh,
    compiler_params=pltpu.CompilerParams(needs_layout_passes=False, use_tc_tiling_on_sc=False),
    scratch_types=[
        pltpu.VMEM((_BPW,), jnp.int32),
        pltpu.VMEM((_BPW,), jnp.int32),
        pltpu.VMEM((_BPW,), jnp.int32),
        pltpu.VMEM((_BPW,), jnp.int32),
        pltpu.VMEM((_BPW, _EMB), jnp.float32),
        pltpu.VMEM((_BPW, _EMB), jnp.float32),
        pltpu.VMEM((_BPW, _EMB), jnp.float32),
        pltpu.VMEM((_BPW, _EMB), jnp.float32),
        pltpu.VMEM((_BPW,), jnp.float32),
        pltpu.SemaphoreType.DMA,
    ],
)


@jax.jit
def kernel(h_idx, r_idx, t_idx, time_idx, entity_emb, relation_emb, time_emb):
  return _sc_call(
      h_idx.astype(jnp.int32), r_idx.astype(jnp.int32),
      t_idx.astype(jnp.int32), time_idx.astype(jnp.int32),
      entity_emb, relation_emb, time_emb)
